# baseline (speedup 1.0000x reference)
import functools

import jax
import jax.numpy as jnp
from jax import lax
from jax.experimental import pallas as pl
from jax.experimental.pallas import tpu as pltpu
from jax.experimental.pallas import tpu_sc as plsc

NC = 2
NS = 16
NW = NC * NS
L = 16
D = 256
P = 25
PPAD = 32
KR = 8
KH = KR * NS


def _body(b_per_w, n_chunk, ch,
          patterns_hbm, scale_hbm, shift_hbm, idx_hbm,
          out_hbm, rep_hbm,
          pat_v, sc_v, sh_v, tab_v, idx_v, rows0, rows1, rows2, rows3,
          gi, g0, g1, g2, g3, p0, p1, p2, p3):
    cid = lax.axis_index("c")
    sid = lax.axis_index("s")
    w = sid * NC + cid
    base = w * b_per_w

    idx_cp = pltpu.make_async_copy(
        idx_hbm.at[pl.ds(base, b_per_w)], idx_v, gi)
    idx_cp.start()
    pltpu.sync_copy(patterns_hbm, pat_v)
    pltpu.sync_copy(scale_hbm, sc_v)
    pltpu.sync_copy(shift_hbm, sh_v)

    def build_row(p, _):
        pv = jnp.full((L,), p, jnp.int32)
        scale = plsc.load_gather(sc_v, [pv])
        shf = plsc.load_gather(sh_v, [pv])
        s = (shf * 10.0).astype(jnp.int32)
        for c in range(D // L):
            col = lax.iota(jnp.int32, L) + (c * L)
            src = lax.rem(col - s, D)
            src = src + jnp.where(src < 0, D, 0)
            vals = plsc.load_gather(pat_v, [pv, src])
            tab_v[p, pl.ds(c * L, L)] = vals * scale
        return 0

    lax.fori_loop(0, P, build_row, 0)

    r0 = (cid * KH + KR * sid) * PPAD
    for k in range(KR):
        pltpu.sync_copy(tab_v, rep_hbm.at[pl.ds(r0 + k * PPAD, PPAD)])

    idx_cp.wait()

    def spread(j, _):
        offs = (cid * KH + ((lax.iota(jnp.int32, L) + j * L) % KH)) * PPAD
        idx_v[pl.ds(j * L, L)] = idx_v[pl.ds(j * L, L)] + offs
        return 0

    lax.fori_loop(0, b_per_w // L, spread, 0)
    plsc.subcore_barrier()

    NB = 4
    rows = (rows0, rows1, rows2, rows3)
    gs = (g0, g1, g2, g3)
    ps = (p0, p1, p2, p3)

    def gath(c, b):
        return pltpu.make_async_copy(
            rep_hbm.at[idx_v.at[pl.ds(c * ch, ch)]], rows[b], gs[b])

    def put(c, b):
        return pltpu.make_async_copy(
            rows[b], out_hbm.at[pl.ds(base + c * ch, ch)], ps[b])

    for b in range(NB):
        gath(b, b).start()

    def body(i, _):
        c0 = NB * i
        for b in range(NB):
            gath(c0 + b, b).wait()
            put(c0 + b, b).start()
        for b in range(NB):
            put(c0 + b, b).wait()

            @pl.when(c0 + b + NB < n_chunk)
            def _():
                gath(c0 + b + NB, b).start()
        return 0

    lax.fori_loop(0, n_chunk // NB, body, 0)


def kernel(indices, patterns, amplitude_scale, frequency_shift):
    bsz, seq = indices.shape
    n = bsz * seq
    b_per_w = n // NW
    ch = 80
    n_chunk = b_per_w // ch

    mesh = plsc.VectorSubcoreMesh(
        core_axis_name="c", subcore_axis_name="s",
        num_cores=NC, num_subcores=NS)

    scale_p = jnp.zeros((PPAD,), jnp.float32).at[:P].set(amplitude_scale)
    shift_p = jnp.zeros((PPAD,), jnp.float32).at[:P].set(frequency_shift)

    run = pl.kernel(
        functools.partial(_body, b_per_w, n_chunk, ch),
        out_type=(
            jax.ShapeDtypeStruct((n, D), jnp.float32),
            jax.ShapeDtypeStruct((NC * KH * PPAD, D), jnp.float32),
        ),
        mesh=mesh,
        compiler_params=pltpu.CompilerParams(needs_layout_passes=False),
        scratch_types=[
            pltpu.VMEM((P, D), jnp.float32),
            pltpu.VMEM((PPAD,), jnp.float32),
            pltpu.VMEM((PPAD,), jnp.float32),
            pltpu.VMEM((PPAD, D), jnp.float32),
            pltpu.VMEM((b_per_w,), jnp.int32),
            pltpu.VMEM((ch, D), jnp.float32),
            pltpu.VMEM((ch, D), jnp.float32),
            pltpu.VMEM((ch, D), jnp.float32),
            pltpu.VMEM((ch, D), jnp.float32),
            pltpu.SemaphoreType.DMA,
            pltpu.SemaphoreType.DMA,
            pltpu.SemaphoreType.DMA,
            pltpu.SemaphoreType.DMA,
            pltpu.SemaphoreType.DMA,
            pltpu.SemaphoreType.DMA,
            pltpu.SemaphoreType.DMA,
            pltpu.SemaphoreType.DMA,
            pltpu.SemaphoreType.DMA,
        ],
    )
    out, _ = run(patterns, scale_p, shift_p, indices.reshape(n))
    return out.reshape(bsz, seq, D)
